# Initial kernel scaffold; baseline (speedup 1.0000x reference)
import jax
import jax.numpy as jnp
from jax.experimental import pallas as pl


def kernel(hidden, W, b, prev_log_probs):
    raise NotImplementedError("write your pallas kernel here")


# speedup vs baseline: 24.1157x; 24.1157x over previous
import functools

import jax
import jax.numpy as jnp
from jax.experimental import pallas as pl
from jax.experimental.pallas import tpu as pltpu

BEAMS = 128
HID = 1024
VOCAB = 100000
K = 8
BV = 2048
NBLK = (VOCAB + BV - 1) // BV

NEG = -1e30
BIGI = 2**30


def _extract_topk(x, ids, k):
    vs, is_ = [], []
    for _ in range(k):
        m = jnp.max(x, axis=1, keepdims=True)
        eq = x == m
        sel = jnp.min(jnp.where(eq, ids, BIGI), axis=1, keepdims=True)
        vs.append(m)
        is_.append(sel)
        x = jnp.where(ids == sel, NEG, x)
    return jnp.concatenate(vs, axis=1), jnp.concatenate(is_, axis=1)


def _step(hid_ref, w_ref, b_ref, prev_ref,
          vals_out, ids_out,
          m_scr, s_scr, tv_scr, ti_scr):
    j = pl.program_id(0)

    @pl.when(j == 0)
    def _init():
        m_scr[...] = jnp.full((BEAMS, 1), NEG, jnp.float32)
        s_scr[...] = jnp.zeros((BEAMS, 1), jnp.float32)
        tv_scr[...] = jnp.full((BEAMS, K), NEG, jnp.float32)
        ti_scr[...] = jnp.zeros((BEAMS, K), jnp.int32)

    x = jax.lax.dot_general(
        hid_ref[...], w_ref[...], (((1,), (0,)), ((), ())),
        preferred_element_type=jnp.float32,
        precision=jax.lax.Precision.HIGHEST,
    ) + b_ref[...]

    col = jax.lax.broadcasted_iota(jnp.int32, (BEAMS, BV), 1) + j * BV
    x = jnp.where(col < VOCAB, x, NEG)

    m_old = m_scr[...]
    bm = jnp.max(x, axis=1, keepdims=True)
    m_new = jnp.maximum(m_old, bm)
    s_scr[...] = (s_scr[...] * jnp.exp(m_old - m_new)
                  + jnp.sum(jnp.exp(x - m_new), axis=1, keepdims=True))
    m_scr[...] = m_new

    bv_, bi_ = _extract_topk(x, col, K)
    allv = jnp.concatenate([tv_scr[...], bv_], axis=1)
    alli = jnp.concatenate([ti_scr[...], bi_], axis=1)
    nv, ni = _extract_topk(allv, alli, K)
    tv_scr[...] = nv
    ti_scr[...] = ni

    @pl.when(j == NBLK - 1)
    def _finalize():
        lse = m_scr[...] + jnp.log(s_scr[...])
        sc = prev_ref[...] + tv_scr[...] - lse
        row = jax.lax.broadcasted_iota(jnp.int32, (BEAMS, K), 0)
        flat = row * VOCAB + ti_scr[...]
        ocol = jax.lax.broadcasted_iota(jnp.int32, (1, K), 1)
        ov = jnp.zeros((1, K), jnp.float32)
        oi = jnp.zeros((1, K), jnp.int32)
        for r in range(K):
            m = jnp.max(sc, axis=(0, 1), keepdims=True)
            chosen = jnp.min(jnp.where(sc == m, flat, BIGI),
                             axis=(0, 1), keepdims=True)
            ov = jnp.where(ocol == r, m, ov)
            oi = jnp.where(ocol == r, chosen, oi)
            sc = jnp.where(flat == chosen, NEG, sc)
        vals_out[...] = ov
        ids_out[...] = oi


@functools.partial(jax.jit, static_argnames=())
def kernel(hidden, W, b, prev_log_probs):
    b2 = b.reshape(1, VOCAB)
    prev2 = prev_log_probs.reshape(BEAMS, 1)
    vals, flat = pl.pallas_call(
        _step,
        grid=(NBLK,),
        in_specs=[
            pl.BlockSpec((BEAMS, HID), lambda j: (0, 0)),
            pl.BlockSpec((HID, BV), lambda j: (0, j)),
            pl.BlockSpec((1, BV), lambda j: (0, j)),
            pl.BlockSpec((BEAMS, 1), lambda j: (0, 0)),
        ],
        out_specs=[
            pl.BlockSpec((1, K), lambda j: (0, 0)),
            pl.BlockSpec((1, K), lambda j: (0, 0)),
        ],
        out_shape=[
            jax.ShapeDtypeStruct((1, K), jnp.float32),
            jax.ShapeDtypeStruct((1, K), jnp.int32),
        ],
        scratch_shapes=[
            pltpu.VMEM((BEAMS, 1), jnp.float32),
            pltpu.VMEM((BEAMS, 1), jnp.float32),
            pltpu.VMEM((BEAMS, K), jnp.float32),
            pltpu.VMEM((BEAMS, K), jnp.int32),
        ],
        compiler_params=pltpu.CompilerParams(
            dimension_semantics=("arbitrary",),
        ),
    )(hidden, W, b2, prev2)
    vals = vals.reshape(K)
    flat = flat.reshape(K)
    beam_ids = flat // VOCAB
    token_ids = flat % VOCAB
    return vals, beam_ids, token_ids


# speedup vs baseline: 31.0475x; 1.2874x over previous
import functools

import jax
import jax.numpy as jnp
from jax.experimental import pallas as pl
from jax.experimental.pallas import tpu as pltpu

BEAMS = 128
HID = 1024
VOCAB = 100000
K = 8
BV = 2048
NBLK = (VOCAB + BV - 1) // BV
LANES = 128
NTILE = BV // LANES

NEG = -1e30
BIGI = 2**30


def _bitonic_merge_desc(xs):
    n = len(xs)
    if n == 1:
        return xs
    half = n // 2
    hi = [jnp.maximum(xs[i], xs[i + half]) for i in range(half)]
    lo = [jnp.minimum(xs[i], xs[i + half]) for i in range(half)]
    return _bitonic_merge_desc(hi) + _bitonic_merge_desc(lo)


def _merge_desc(a, b):
    return _bitonic_merge_desc(a + b[::-1])


def _merge_top8(a, b):
    m = [jnp.maximum(a[i], b[7 - i]) for i in range(8)]
    return _bitonic_merge_desc(m)


def _bitonic_merge_desc_kv(vs, ids):
    n = len(vs)
    if n == 1:
        return vs, ids
    half = n // 2
    hv, hi, lv, li = [], [], [], []
    for i in range(half):
        av, ai, bv, bi = vs[i], ids[i], vs[i + half], ids[i + half]
        c = (bv > av) | ((bv == av) & (bi < ai))
        hv.append(jnp.where(c, bv, av))
        hi.append(jnp.where(c, bi, ai))
        lv.append(jnp.where(c, av, bv))
        li.append(jnp.where(c, ai, bi))
    rhv, rhi = _bitonic_merge_desc_kv(hv, hi)
    rlv, rli = _bitonic_merge_desc_kv(lv, li)
    return rhv + rlv, rhi + rli


def _merge_top8_kv(av, ai, bv, bi):
    mv, mi = [], []
    for i in range(8):
        x, xi_, y, yi = av[i], ai[i], bv[7 - i], bi[7 - i]
        c = (y > x) | ((y == x) & (yi < xi_))
        mv.append(jnp.where(c, y, x))
        mi.append(jnp.where(c, yi, xi_))
    return _bitonic_merge_desc_kv(mv, mi)


def _extract_topk(x, ids, k):
    vs, is_ = [], []
    for _ in range(k):
        m = jnp.max(x, axis=1, keepdims=True)
        sel = jnp.min(jnp.where(x == m, ids, BIGI), axis=1, keepdims=True)
        vs.append(m)
        is_.append(sel)
        x = jnp.where(ids == sel, NEG, x)
    return jnp.concatenate(vs, axis=1), jnp.concatenate(is_, axis=1)


def _step(hid_ref, w_ref, b_ref, prev_ref,
          vals_out, ids_out,
          m_scr, s_scr, pv_scr, pi_scr):
    j = pl.program_id(0)

    @pl.when(j == 0)
    def _init():
        m_scr[...] = jnp.full((BEAMS, 1), NEG, jnp.float32)
        s_scr[...] = jnp.zeros((BEAMS, 1), jnp.float32)
        pv_scr[...] = jnp.full((BEAMS, K * LANES), NEG, jnp.float32)
        pi_scr[...] = jnp.full((BEAMS, K * LANES), BIGI, jnp.int32)

    x = jax.lax.dot_general(
        hid_ref[...], w_ref[...], (((1,), (0,)), ((), ())),
        preferred_element_type=jnp.float32,
        precision=jax.lax.Precision.HIGHEST,
    ) + b_ref[...]

    col = jax.lax.broadcasted_iota(jnp.int32, (BEAMS, BV), 1) + j * BV
    x = jnp.where(col < VOCAB, x, NEG)

    m_old = m_scr[...]
    bm = jnp.max(x, axis=1, keepdims=True)
    m_new = jnp.maximum(m_old, bm)
    s_scr[...] = (s_scr[...] * jnp.exp(m_old - m_new)
                  + jnp.sum(jnp.exp(x - m_new), axis=1, keepdims=True))
    m_scr[...] = m_new

    tiles = []
    for i in range(NTILE):
        xi = jax.lax.bitcast_convert_type(
            x[:, i * LANES:(i + 1) * LANES], jnp.int32)
        tiles.append([jax.lax.bitcast_convert_type((xi & -16) | i,
                                                   jnp.float32)])

    while len(tiles) > 2:
        tiles = [_merge_desc(tiles[t], tiles[t + 1])
                 for t in range(0, len(tiles), 2)]
    blk = _merge_top8(tiles[0], tiles[1])

    lane = jax.lax.broadcasted_iota(jnp.int32, (BEAMS, LANES), 1)
    bv_, bi_ = [], []
    for s in range(K):
        y = jax.lax.bitcast_convert_type(blk[s], jnp.int32)
        bi_.append((y & 15) * LANES + lane + j * BV)
        bv_.append(jax.lax.bitcast_convert_type(y & -16, jnp.float32))

    pv = [pv_scr[:, p * LANES:(p + 1) * LANES] for p in range(K)]
    pi = [pi_scr[:, p * LANES:(p + 1) * LANES] for p in range(K)]
    nv, ni = _merge_top8_kv(pv, pi, bv_, bi_)
    for p in range(K):
        pv_scr[:, p * LANES:(p + 1) * LANES] = nv[p]
        pi_scr[:, p * LANES:(p + 1) * LANES] = ni[p]

    @pl.when(j == NBLK - 1)
    def _finalize():
        tv, ti = _extract_topk(pv_scr[...], pi_scr[...], K)
        lse = m_scr[...] + jnp.log(s_scr[...])
        sc = prev_ref[...] + tv - lse
        row = jax.lax.broadcasted_iota(jnp.int32, (BEAMS, K), 0)
        flat = row * VOCAB + ti
        ocol = jax.lax.broadcasted_iota(jnp.int32, (1, K), 1)
        ov = jnp.zeros((1, K), jnp.float32)
        oi = jnp.zeros((1, K), jnp.int32)
        for r in range(K):
            m = jnp.max(sc, axis=(0, 1), keepdims=True)
            chosen = jnp.min(jnp.where(sc == m, flat, BIGI),
                             axis=(0, 1), keepdims=True)
            ov = jnp.where(ocol == r, m, ov)
            oi = jnp.where(ocol == r, chosen, oi)
            sc = jnp.where(flat == chosen, NEG, sc)
        vals_out[...] = ov
        ids_out[...] = oi


@functools.partial(jax.jit, static_argnames=())
def kernel(hidden, W, b, prev_log_probs):
    b2 = b.reshape(1, VOCAB)
    prev2 = prev_log_probs.reshape(BEAMS, 1)
    vals, flat = pl.pallas_call(
        _step,
        grid=(NBLK,),
        in_specs=[
            pl.BlockSpec((BEAMS, HID), lambda j: (0, 0)),
            pl.BlockSpec((HID, BV), lambda j: (0, j)),
            pl.BlockSpec((1, BV), lambda j: (0, j)),
            pl.BlockSpec((BEAMS, 1), lambda j: (0, 0)),
        ],
        out_specs=[
            pl.BlockSpec((1, K), lambda j: (0, 0)),
            pl.BlockSpec((1, K), lambda j: (0, 0)),
        ],
        out_shape=[
            jax.ShapeDtypeStruct((1, K), jnp.float32),
            jax.ShapeDtypeStruct((1, K), jnp.int32),
        ],
        scratch_shapes=[
            pltpu.VMEM((BEAMS, 1), jnp.float32),
            pltpu.VMEM((BEAMS, 1), jnp.float32),
            pltpu.VMEM((BEAMS, K * LANES), jnp.float32),
            pltpu.VMEM((BEAMS, K * LANES), jnp.int32),
        ],
        compiler_params=pltpu.CompilerParams(
            dimension_semantics=("arbitrary",),
        ),
    )(hidden, W, b2, prev2)
    vals = vals.reshape(K)
    flat = flat.reshape(K)
    beam_ids = flat // VOCAB
    token_ids = flat % VOCAB
    return vals, beam_ids, token_ids
